# baseline (speedup 1.0000x reference)
import functools

import jax
import jax.numpy as jnp
from jax import lax
from jax.experimental import pallas as pl
from jax.experimental.pallas import tpu as pltpu
from jax.experimental.pallas import tpu_sc as plsc

_ROWS = 128
_LANES = 16


def _sc_lookup(tokens, table_flat, B, C, T, V, D):
    t_half = T // 2
    n_flat = t_half * C
    n_it = n_flat // _ROWS
    tok_per_it = _ROWS // C

    mesh = plsc.VectorSubcoreMesh(core_axis_name="c", subcore_axis_name="s")

    @functools.partial(
        pl.kernel,
        mesh=mesh,
        out_type=jax.ShapeDtypeStruct((B, 2, n_flat, D), jnp.float32),
        scratch_types=[
            pltpu.VMEM((C, t_half), jnp.int32),
            pltpu.VMEM((n_flat,), jnp.int32),
            pltpu.VMEM((6, _ROWS, D), jnp.float32),
            pltpu.SemaphoreType.DMA,
            pltpu.SemaphoreType.DMA,
            pltpu.SemaphoreType.DMA,
            pltpu.SemaphoreType.DMA,
            pltpu.SemaphoreType.DMA,
            pltpu.SemaphoreType.DMA,
            pltpu.SemaphoreType.DMA,
            pltpu.SemaphoreType.DMA,
            pltpu.SemaphoreType.DMA,
            pltpu.SemaphoreType.DMA,
            pltpu.SemaphoreType.DMA,
            pltpu.SemaphoreType.DMA,
        ],
        compiler_params=pltpu.CompilerParams(
            use_tc_tiling_on_sc=False, needs_layout_passes=False
        ),
    )
    def body(
        tokens_hbm, table_hbm, out_hbm, raw_v, idx_v, rows_v,
        g0, g1, g2, g3, g4, g5, w0, w1, w2, w3, w4, w5,
    ):
        b = lax.axis_index("s")
        half = lax.axis_index("c")
        g_sem = (g0, g1, g2, g3, g4, g5)
        w_sem = (w0, w1, w2, w3, w4, w5)

        pltpu.sync_copy(tokens_hbm.at[b, :, pl.ds(half * t_half, t_half)], raw_v)
        lane = lax.broadcasted_iota(jnp.int32, (_LANES,), 0)

        n_tb = t_half // _LANES
        lane_c = lane * C

        def transpose_fold(tb, carry):
            t0 = tb * _LANES
            base = lane_c + t0 * C
            for i in range(C):
                vals = raw_v[i, pl.ds(t0, _LANES)] + i * V
                plsc.store_scatter(idx_v, [base + i], vals)
            return carry

        lax.fori_loop(0, n_tb, transpose_fold, 0)

        def gather(it, p):
            return pltpu.make_async_copy(
                table_hbm.at[idx_v.at[pl.ds(it * _ROWS, _ROWS)]],
                rows_v.at[p],
                g_sem[p],
            )

        def write(it, p):
            return pltpu.make_async_copy(
                rows_v.at[p],
                out_hbm.at[b, half, pl.ds(it * _ROWS, _ROWS)],
                w_sem[p],
            )

        _DEPTH = 6

        def stage(it, p, q):
            @pl.when(it >= 1)
            def _():
                write(it - 1, q).wait()

            @pl.when(it + _DEPTH - 1 < n_it)
            def _():
                gather(it + _DEPTH - 1, q).start()

            gather(it, p).wait()
            write(it, p).start()

        for k in range(_DEPTH - 1):
            gather(k, k).start()

        def rounds(itd, carry):
            for r in range(_DEPTH):
                stage(_DEPTH * itd + r, r, (r + _DEPTH - 1) % _DEPTH)
            return carry

        n_rounds = n_it // _DEPTH
        lax.fori_loop(0, n_rounds, rounds, 0)
        for it in range(n_rounds * _DEPTH, n_it):
            stage(it, it % _DEPTH, (it + _DEPTH - 1) % _DEPTH)
        write(n_it - 1, (n_it - 1) % _DEPTH).wait()

    return body(tokens, table_flat)


def kernel(tokens, tables):
    B, C, T = tokens.shape
    C2, V, D = tables.shape
    assert C == C2
    table_flat = tables.reshape(C * V, D)
    out = _sc_lookup(tokens.astype(jnp.int32), table_flat, B, C, T, V, D)
    return out.reshape(B, T, C * D)


# speedup vs baseline: 1.0010x; 1.0010x over previous
import functools

import jax
import jax.numpy as jnp
from jax import lax
from jax.experimental import pallas as pl
from jax.experimental.pallas import tpu as pltpu
from jax.experimental.pallas import tpu_sc as plsc

_ROWS = 128
_LANES = 16


def _sc_lookup(tokens, table_flat, B, C, T, V, D):
    t_half = T // 2
    n_flat = t_half * C
    n_it = n_flat // _ROWS

    mesh = plsc.VectorSubcoreMesh(core_axis_name="c", subcore_axis_name="s")

    @functools.partial(
        pl.kernel,
        mesh=mesh,
        out_type=jax.ShapeDtypeStruct((B, 2, n_flat, D), jnp.float32),
        scratch_types=[
            pltpu.VMEM((C, t_half), jnp.int32),
            pltpu.VMEM((n_flat,), jnp.int32),
            pltpu.VMEM((6, _ROWS, D), jnp.float32),
            pltpu.SemaphoreType.DMA,
            pltpu.SemaphoreType.DMA,
            pltpu.SemaphoreType.DMA,
            pltpu.SemaphoreType.DMA,
            pltpu.SemaphoreType.DMA,
            pltpu.SemaphoreType.DMA,
            pltpu.SemaphoreType.DMA,
            pltpu.SemaphoreType.DMA,
            pltpu.SemaphoreType.DMA,
            pltpu.SemaphoreType.DMA,
            pltpu.SemaphoreType.DMA,
            pltpu.SemaphoreType.DMA,
        ],
        compiler_params=pltpu.CompilerParams(
            use_tc_tiling_on_sc=False, needs_layout_passes=False
        ),
    )
    def body(
        tokens_hbm, table_hbm, out_hbm, raw_v, idx_v, rows_v,
        g0, g1, g2, g3, g4, g5, w0, w1, w2, w3, w4, w5,
    ):
        b = lax.axis_index("s")
        half = lax.axis_index("c")
        g_sem = (g0, g1, g2, g3, g4, g5)
        w_sem = (w0, w1, w2, w3, w4, w5)

        pltpu.sync_copy(tokens_hbm.at[b, :, pl.ds(half * t_half, t_half)], raw_v)
        lane = lax.broadcasted_iota(jnp.int32, (_LANES,), 0)

        n_tb = t_half // _LANES
        lane_c = lane * C

        def transpose_fold(tb, carry):
            t0 = tb * _LANES
            base = lane_c + t0 * C
            for i in range(C):
                vals = raw_v[i, pl.ds(t0, _LANES)] + i * V
                plsc.store_scatter(idx_v, [base + i], vals)
            return carry

        lax.fori_loop(0, n_tb, transpose_fold, 0)

        def gather(it, p):
            return pltpu.make_async_copy(
                table_hbm.at[idx_v.at[pl.ds(it * _ROWS, _ROWS)]],
                rows_v.at[p],
                g_sem[p],
            )

        def write(it, p):
            return pltpu.make_async_copy(
                rows_v.at[p],
                out_hbm.at[b, half, pl.ds(it * _ROWS, _ROWS)],
                w_sem[p],
            )

        _DEPTH = 6

        def stage(it, p, q):
            @pl.when(it >= 1)
            def _():
                write(it - 1, q).wait()

            @pl.when(it + _DEPTH - 1 < n_it)
            def _():
                gather(it + _DEPTH - 1, q).start()

            gather(it, p).wait()
            write(it, p).start()

        for k in range(_DEPTH - 1):
            gather(k, k).start()

        def rounds(itd, carry):
            for r in range(_DEPTH):
                stage(_DEPTH * itd + r, r, (r + _DEPTH - 1) % _DEPTH)
            return carry

        n_rounds = n_it // _DEPTH
        lax.fori_loop(0, n_rounds, rounds, 0)
        for it in range(n_rounds * _DEPTH, n_it):
            stage(it, it % _DEPTH, (it + _DEPTH - 1) % _DEPTH)
        write(n_it - 1, (n_it - 1) % _DEPTH).wait()

    return body(tokens, table_flat)


def kernel(tokens, tables):
    B, C, T = tokens.shape
    C2, V, D = tables.shape
    assert C == C2
    table_flat = tables.reshape(C * V, D)
    out = _sc_lookup(tokens.astype(jnp.int32), table_flat, B, C, T, V, D)
    return out.reshape(B, T, C * D)
